# baseline (speedup 1.0000x reference)
import functools

import jax
import jax.numpy as jnp
from jax import lax
from jax.experimental import pallas as pl
from jax.experimental.pallas import tpu as pltpu
from jax.experimental.pallas import tpu_sc as plsc

CH = 32


def _pack_sc(xf, B, L, d):
    info = plsc.get_sparse_core_info()
    NC, NS, NL = info.num_cores, info.num_subcores, info.num_lanes
    NW = NC * NS
    R = (L * B) // NW
    n_chunks = R // CH
    n_groups = R // NL + CH // NL

    mesh = plsc.VectorSubcoreMesh(core_axis_name="c", subcore_axis_name="s")

    @functools.partial(
        pl.kernel,
        mesh=mesh,
        out_type=jax.ShapeDtypeStruct((L * B, d), jnp.float32),
        scratch_types=[
            pltpu.VMEM((R + CH,), jnp.int32),
            pltpu.VMEM((CH, d), jnp.float32),
            pltpu.VMEM((CH, d), jnp.float32),
            pltpu.SemaphoreType.DMA,
            pltpu.SemaphoreType.DMA,
            pltpu.SemaphoreType.DMA,
            pltpu.SemaphoreType.DMA,
        ],
    )
    def k(x_hbm, out_hbm, idxv, buf0, buf1, rsem0, rsem1, wsem0, wsem1):
        wid = lax.axis_index("s") * NC + lax.axis_index("c")
        base = wid * R
        tbase = base // B

        def gen(j, _):
            t = jnp.minimum(tbase + j, L - 1)
            idxv[pl.ds(NL * j, NL)] = lax.iota(jnp.int32, NL) * L + t
            return _

        lax.fori_loop(0, n_groups, gen, None)

        def gather(c, buf, sem):
            return pltpu.make_async_copy(
                x_hbm.at[idxv.at[pl.ds(c * CH, CH)]], buf, sem)

        def put(c, buf, sem):
            return pltpu.make_async_copy(
                buf, out_hbm.at[pl.ds(base + c * CH, CH)], sem)

        gather(0, buf0, rsem0).start()

        def step(j, _):
            c0 = 2 * j
            c1 = c0 + 1
            gather(c1, buf1, rsem1).start()
            gather(c0, buf0, rsem0).wait()
            w0 = put(c0, buf0, wsem0)
            w0.start()
            w0.wait()
            gather(c0 + 2, buf0, rsem0).start()
            gather(c1, buf1, rsem1).wait()
            w1 = put(c1, buf1, wsem1)
            w1.start()
            w1.wait()
            return _

        lax.fori_loop(0, n_chunks // 2, step, None)
        gather(n_chunks, buf0, rsem0).wait()

    return k(xf)


def kernel(x):
    B, L, d = x.shape
    data = _pack_sc(x.reshape(B * L, d), B, L, d)
    batch_sizes = jnp.full((L,), B, dtype=jnp.int64)
    sorted_indices = jnp.arange(B, dtype=jnp.int64)
    unsorted_indices = jnp.arange(B, dtype=jnp.int64)
    return data, batch_sizes, sorted_indices, unsorted_indices
